# Initial kernel scaffold; baseline (speedup 1.0000x reference)
import jax
import jax.numpy as jnp
from jax.experimental import pallas as pl


def kernel(x0, x1, x2, edge_index0, edge_index1, edge_index2, W1, b1, Wz, bz, LzW, Lzb, Wr, br, LrW, Lrb, Wh, bh, LhW, Lhb, W2, b2):
    raise NotImplementedError("write your pallas kernel here")


# speedup vs baseline: 23.0786x; 23.0786x over previous
import functools

import jax
import jax.numpy as jnp
from jax import lax
from jax.experimental import pallas as pl
from jax.experimental.pallas import tpu as pltpu
from jax.experimental.pallas import tpu_sc as plsc

N = 10000
E = 320000
D = 128
D_OUT = 64

NC = 2
NS = 16
NW = NC * NS
EPT = E // NW
CH = 80
NCHUNK = EPT // CH
RPS = 624
ZR = 208
BN = 2000
DEGW = 1

_mesh = lambda: plsc.VectorSubcoreMesh(core_axis_name="c", subcore_axis_name="s")


DR = 80


def _sc_degree(dst0, dst1, dst2):
    @functools.partial(
        pl.kernel,
        mesh=_mesh(),
        out_type=jax.ShapeDtypeStruct((3, NC, DR, D), jnp.float32),
        scratch_types=[
            pltpu.VMEM((EPT,), jnp.int32),
            pltpu.VMEM((DR, D), jnp.float32),
            pltpu.VMEM((DR,), jnp.int32),
            pltpu.VMEM((16, D), jnp.float32),
            pltpu.VMEM_SHARED((DR, D), jnp.float32),
        ],
        compiler_params=pltpu.CompilerParams(needs_layout_passes=False),
    )
    def k(d0, d1, d2, out, dstv, degv, iotav, zrows, degsh):
        c = lax.axis_index("c")
        s = lax.axis_index("s")
        wid = s * NC + c
        ones = jnp.ones((16,), jnp.float32)
        zero16 = jnp.zeros((16,), jnp.float32)

        for b in range(DR // 16):
            iotav[pl.ds(b * 16, 16)] = lax.iota(jnp.int32, 16) + b * 16

        def zr(i, carry):
            for j in range(D // 16):
                zrows[i, pl.ds(j * 16, 16)] = zero16
            return carry
        lax.fori_loop(0, 16, zr, 0)

        for t, dh in enumerate((d0, d1, d2)):
            @pl.when(s == 0)
            def _():
                for b in range(DR // 16):
                    pltpu.sync_copy(zrows, degsh.at[pl.ds(b * 16, 16)])

            def zb(i, carry):
                for j in range(D // 16):
                    degv[i, pl.ds(j * 16, 16)] = zero16
                return carry
            lax.fori_loop(0, DR, zb, 0)
            pltpu.sync_copy(dh.at[pl.ds(wid * EPT, EPT)], dstv)

            def ab(i, carry):
                idx = dstv[pl.ds(pl.multiple_of(i * 16, 16), 16)]
                plsc.addupdate_scatter(
                    degv,
                    [lax.shift_right_logical(idx, 7), lax.bitwise_and(idx, 127)],
                    ones)
                return carry
            lax.fori_loop(0, EPT // 16, ab, 0)
            plsc.subcore_barrier()
            pltpu.sync_copy(degv, degsh.at[iotav], add=True)
            plsc.subcore_barrier()

            @pl.when(s < 5)
            def _():
                r0 = pl.multiple_of(s * 16, 16)
                pltpu.sync_copy(degsh.at[pl.ds(r0, 16)],
                                out.at[t, c, pl.ds(r0, 16)])
            plsc.subcore_barrier()

    return k(dst0, dst1, dst2)


def _sc_aggregate(y0, y1, y2, src0, src1, src2, dst0, dst1, dst2):
    @functools.partial(
        pl.kernel,
        mesh=_mesh(),
        out_type=[jax.ShapeDtypeStruct((NC, N, D), jnp.float32)] * 3,
        scratch_types=[
            pltpu.VMEM((CH,), jnp.int32),
            pltpu.VMEM((CH,), jnp.int32),
            pltpu.VMEM((CH, D), jnp.float32),
            pltpu.VMEM((ZR, D), jnp.float32),
            pltpu.VMEM_SHARED((N, D), jnp.float32),
            pltpu.SemaphoreType.DMA,
        ],
        compiler_params=pltpu.CompilerParams(needs_layout_passes=False),
    )
    def k(y0h, y1h, y2h, s0h, s1h, s2h, d0h, d1h, d2h, o0, o1, o2,
          srcv, dstv, rows, zbuf, aggsh, sem):
        c = lax.axis_index("c")
        s = lax.axis_index("s")
        wid = s * NC + c
        z16 = jnp.zeros((16,), jnp.float32)

        def zb(i, carry):
            for j in range(D // 16):
                zbuf[i, pl.ds(j * 16, 16)] = z16
            return carry
        lax.fori_loop(0, ZR, zb, 0)

        for t in range(3):
            yh = (y0h, y1h, y2h)[t]
            eh = (s0h, s1h, s2h)[t]
            dh = (d0h, d1h, d2h)[t]
            oh = (o0, o1, o2)[t]
            for b in range(RPS // ZR):
                pltpu.sync_copy(zbuf, aggsh.at[pl.ds(s * RPS + b * ZR, ZR)])

            @pl.when(s == NS - 1)
            def _():
                pltpu.sync_copy(zbuf.at[pl.ds(0, 16)],
                                aggsh.at[pl.ds(NS * RPS, 16)])
            plsc.subcore_barrier()

            def cbody(ci, carry):
                base = pl.multiple_of(wid * EPT + ci * CH, 8)
                pltpu.sync_copy(eh.at[pl.ds(base, CH)], srcv)
                pltpu.sync_copy(dh.at[pl.ds(base, CH)], dstv)
                pltpu.async_copy(yh.at[srcv], rows, sem).wait()
                pltpu.sync_copy(rows, aggsh.at[dstv], add=True)
                return carry
            lax.fori_loop(0, NCHUNK, cbody, 0)
            plsc.subcore_barrier()
            for b in range(RPS // ZR):
                r0 = s * RPS + b * ZR
                pltpu.sync_copy(aggsh.at[pl.ds(r0, ZR)], oh.at[c, pl.ds(r0, ZR)])

            @pl.when(s == NS - 1)
            def _():
                pltpu.sync_copy(aggsh.at[pl.ds(NS * RPS, 16)],
                                oh.at[c, pl.ds(NS * RPS, 16)])
            plsc.subcore_barrier()

    return k(y0, y1, y2, src0, src1, src2, dst0, dst1, dst2)


def _tc_fuse(Wz, Wr, Wh, Lz1, Lr1, Lh1, bz, br, bh, lzb, lrb, lhb):
    def body(wz, wr, wh, l1z, l1r, l1h, bz_, br_, bh_, lz, lr, lh, wf, bf):
        for g, (w, l1, b_, lb) in enumerate(
                ((wz, l1z, bz_, lz), (wr, l1r, br_, lr), (wh, l1h, bh_, lh))):
            wf[g] = jnp.dot(w[...], l1[...], preferred_element_type=jnp.float32)
            bf[g] = jnp.dot(b_[...], l1[...], preferred_element_type=jnp.float32) + lb[...]

    full = lambda shp: pl.BlockSpec(shp, lambda: tuple(0 for _ in shp))
    return pl.pallas_call(
        body,
        grid=(),
        in_specs=[full((D, D))] * 6 + [full((1, D))] * 6,
        out_specs=[full((3, D, D)), full((3, 1, D))],
        out_shape=[jax.ShapeDtypeStruct((3, D, D), jnp.float32),
                   jax.ShapeDtypeStruct((3, 1, D), jnp.float32)],
    )(Wz, Wr, Wh, Lz1, Lr1, Lh1, bz, br, bh, lzb, lrb, lhb)


def _tc_lin1(x0, x1, x2, degp, W1, b1):
    def body(x0b, x1b, x2b, dgb, w1, b1b, y0b, y1b, y2b, disb):
        w = w1[...]
        for t, (xb, yb) in enumerate(((x0b, y0b), (x1b, y1b), (x2b, y2b))):
            deg = dgb[t, 0, :, 0] + dgb[t, 1, :, 0] + 1.0
            dv = lax.rsqrt(deg)
            disb[t] = dv[:, None]
            yb[...] = (jnp.dot(xb[...], w, preferred_element_type=jnp.float32)
                       + b1b[...]) * dv[:, None]

    row = pl.BlockSpec((BN, D), lambda i: (i, 0))
    return pl.pallas_call(
        body,
        grid=(N // BN,),
        in_specs=[row, row, row,
                  pl.BlockSpec((3, NC, BN, DEGW), lambda i: (0, 0, i, 0)),
                  pl.BlockSpec((D, D), lambda i: (0, 0)),
                  pl.BlockSpec((1, D), lambda i: (0, 0))],
        out_specs=[row, row, row, pl.BlockSpec((3, BN, 1), lambda i: (0, i, 0))],
        out_shape=[jax.ShapeDtypeStruct((N, D), jnp.float32)] * 3
                  + [jax.ShapeDtypeStruct((3, N, 1), jnp.float32)],
    )(x0, x1, x2, degp, W1, b1)


def _tc_cell(st, yt, dist, H, Wf, L2, bf):
    def body(sb, yb, db, hb, wf, l2, bfb, ob):
        agg = (sb[0] + sb[1] + yb[...]) * db[...]
        h = hb[...]
        z = jax.nn.sigmoid(jnp.dot(agg, wf[0], preferred_element_type=jnp.float32)
                           + jnp.dot(h, l2[0], preferred_element_type=jnp.float32)
                           + bfb[0])
        r = jax.nn.sigmoid(jnp.dot(agg, wf[1], preferred_element_type=jnp.float32)
                           + jnp.dot(h, l2[1], preferred_element_type=jnp.float32)
                           + bfb[1])
        ht = jnp.tanh(jnp.dot(agg, wf[2], preferred_element_type=jnp.float32)
                      + jnp.dot(h * r, l2[2], preferred_element_type=jnp.float32)
                      + bfb[2])
        ob[...] = z * h + (1.0 - z) * ht

    row = pl.BlockSpec((BN, D), lambda i: (i, 0))
    return pl.pallas_call(
        body,
        grid=(N // BN,),
        in_specs=[pl.BlockSpec((NC, BN, D), lambda i: (0, i, 0)),
                  row,
                  pl.BlockSpec((BN, 1), lambda i: (i, 0)),
                  row,
                  pl.BlockSpec((3, D, D), lambda i: (0, 0, 0)),
                  pl.BlockSpec((3, D, D), lambda i: (0, 0, 0)),
                  pl.BlockSpec((3, 1, D), lambda i: (0, 0, 0))],
        out_specs=row,
        out_shape=jax.ShapeDtypeStruct((N, D), jnp.float32),
    )(st, yt, dist, H, Wf, L2, bf)


def _tc_pool(H, W2, b2):
    def body(hb, w2, b2b, ob):
        p = jnp.max(hb[...], axis=0, keepdims=True)
        ob[...] = jnp.dot(p, w2[...], preferred_element_type=jnp.float32) + b2b[...]

    full = lambda shp: pl.BlockSpec(shp, lambda: tuple(0 for _ in shp))
    return pl.pallas_call(
        body,
        grid=(),
        in_specs=[full((N, D)), full((D, D_OUT)), full((1, D_OUT))],
        out_specs=full((1, D_OUT)),
        out_shape=jax.ShapeDtypeStruct((1, D_OUT), jnp.float32),
    )(H, W2, b2)


def kernel(x0, x1, x2, edge_index0, edge_index1, edge_index2,
           W1, b1, Wz, bz, LzW, Lzb, Wr, br, LrW, Lrb, Wh, bh, LhW, Lhb, W2, b2):
    srcs = [e[0] for e in (edge_index0, edge_index1, edge_index2)]
    dsts = [e[1] for e in (edge_index0, edge_index1, edge_index2)]

    Wf, bf = _tc_fuse(Wz, Wr, Wh, LzW[:D], LrW[:D], LhW[:D],
                      bz.reshape(1, D), br.reshape(1, D), bh.reshape(1, D),
                      Lzb.reshape(1, D), Lrb.reshape(1, D), Lhb.reshape(1, D))
    L2 = jnp.stack([LzW[D:], LrW[D:], LhW[D:]])

    degp = _sc_degree(*dsts).reshape(3, NC, DR * D, 1)[:, :, :N, :]
    y0, y1, y2, dis = _tc_lin1(x0, x1, x2, degp, W1, b1.reshape(1, D))
    s0, s1, s2 = _sc_aggregate(y0, y1, y2, *srcs, *dsts)

    H = jnp.zeros((N, D), jnp.float32)
    for t, (st, yt) in enumerate(((s0, y0), (s1, y1), (s2, y2))):
        H = _tc_cell(st, yt, dis[t], H, Wf, L2, bf)
    return _tc_pool(H, W2, b2.reshape(1, D_OUT))


# speedup vs baseline: 38.8772x; 1.6846x over previous
import functools

import jax
import jax.numpy as jnp
from jax import lax
from jax.experimental import pallas as pl
from jax.experimental.pallas import tpu as pltpu
from jax.experimental.pallas import tpu_sc as plsc

N = 10000
E = 320000
D = 128
D_OUT = 64

NC = 2
NS = 16
NW = NC * NS
EPT = E // NW
CH = 80
NCHUNK = EPT // CH
RPS = 624
ZR = 208
ZB = 16
BN = 2000
DEGW = 1

_mesh = lambda: plsc.VectorSubcoreMesh(core_axis_name="c", subcore_axis_name="s")


DR = 80


def _sc_degree(dst0, dst1, dst2):
    @functools.partial(
        pl.kernel,
        mesh=_mesh(),
        out_type=jax.ShapeDtypeStruct((3, NC, DR, D), jnp.float32),
        scratch_types=[
            pltpu.VMEM((EPT,), jnp.int32),
            pltpu.VMEM((DR, D), jnp.float32),
            pltpu.VMEM((DR,), jnp.int32),
            pltpu.VMEM((16, D), jnp.float32),
            pltpu.VMEM_SHARED((DR, D), jnp.float32),
        ],
        compiler_params=pltpu.CompilerParams(needs_layout_passes=False),
    )
    def k(d0, d1, d2, out, dstv, degv, iotav, zrows, degsh):
        c = lax.axis_index("c")
        s = lax.axis_index("s")
        wid = s * NC + c
        ones = jnp.ones((16,), jnp.float32)
        zero16 = jnp.zeros((16,), jnp.float32)

        for b in range(DR // 16):
            iotav[pl.ds(b * 16, 16)] = lax.iota(jnp.int32, 16) + b * 16

        def zr(i, carry):
            for j in range(D // 16):
                zrows[i, pl.ds(j * 16, 16)] = zero16
            return carry
        lax.fori_loop(0, 16, zr, 0)

        for t, dh in enumerate((d0, d1, d2)):
            @pl.when(s == 0)
            def _():
                for b in range(DR // 16):
                    pltpu.sync_copy(zrows, degsh.at[pl.ds(b * 16, 16)])

            def zb(i, carry):
                for j in range(D // 16):
                    degv[i, pl.ds(j * 16, 16)] = zero16
                return carry
            lax.fori_loop(0, DR, zb, 0)
            pltpu.sync_copy(dh.at[pl.ds(wid * EPT, EPT)], dstv)

            def ab(i, carry):
                idx = dstv[pl.ds(pl.multiple_of(i * 16, 16), 16)]
                plsc.addupdate_scatter(
                    degv,
                    [lax.shift_right_logical(idx, 7), lax.bitwise_and(idx, 127)],
                    ones)
                return carry
            lax.fori_loop(0, EPT // 16, ab, 0)
            plsc.subcore_barrier()
            pltpu.sync_copy(degv, degsh.at[iotav], add=True)
            plsc.subcore_barrier()

            @pl.when(s < 5)
            def _():
                r0 = pl.multiple_of(s * 16, 16)
                pltpu.sync_copy(degsh.at[pl.ds(r0, 16)],
                                out.at[t, c, pl.ds(r0, 16)])
            plsc.subcore_barrier()

    return k(dst0, dst1, dst2)


def _sc_aggregate(y0, y1, y2, src0, src1, src2, dst0, dst1, dst2):
    @functools.partial(
        pl.kernel,
        mesh=_mesh(),
        out_type=[jax.ShapeDtypeStruct((NC, N, D), jnp.float32)] * 3,
        scratch_types=[
            pltpu.VMEM((EPT,), jnp.int32),
            pltpu.VMEM((NCHUNK, CH), jnp.int32),
            pltpu.VMEM((2, CH, D), jnp.float32),
            pltpu.VMEM((ZB, D), jnp.float32),
            pltpu.VMEM_SHARED((N, D), jnp.float32),
            pltpu.SemaphoreType.DMA,
            pltpu.SemaphoreType.DMA,
        ],
        compiler_params=pltpu.CompilerParams(needs_layout_passes=False),
    )
    def k(y0h, y1h, y2h, s0h, s1h, s2h, d0h, d1h, d2h, o0, o1, o2,
          srcv, dstv2, rows, zbuf, aggsh, sem0, sem1):
        c = lax.axis_index("c")
        s = lax.axis_index("s")
        wid = s * NC + c
        z16 = jnp.zeros((16,), jnp.float32)
        sems = (sem0, sem1)

        def zb(i, carry):
            for j in range(D // 16):
                zbuf[i, pl.ds(j * 16, 16)] = z16
            return carry
        lax.fori_loop(0, ZB, zb, 0)
        nzb = (RPS // ZB) + jnp.where(s == NS - 1, 1, 0)

        for t in range(3):
            yh = (y0h, y1h, y2h)[t]
            eh = (s0h, s1h, s2h)[t]
            dh = (d0h, d1h, d2h)[t]
            oh = (o0, o1, o2)[t]

            pltpu.sync_copy(eh.at[pl.ds(wid * EPT, EPT)], srcv)
            pltpu.sync_copy(dh.at[wid], dstv2)

            def zrow(i, carry):
                pltpu.sync_copy(zbuf, aggsh.at[pl.ds(s * RPS + i * ZB, ZB)])
                return carry
            lax.fori_loop(0, nzb, zrow, 0)
            plsc.subcore_barrier()

            def gather(buf, ci):
                idx = srcv.at[pl.ds(pl.multiple_of(ci * CH, 8), CH)]
                return pltpu.make_async_copy(yh.at[idx], rows.at[buf], sems[buf])

            def scatter(buf, ci):
                pltpu.sync_copy(rows.at[buf], aggsh.at[dstv2.at[ci]], add=True)

            gather(0, 0).start()

            def pair(kk, carry):
                ci0 = kk * 2
                gather(0, ci0).wait()
                gather(1, ci0 + 1).start()
                scatter(0, ci0)
                gather(1, ci0 + 1).wait()
                gather(0, ci0 + 2).start()
                scatter(1, ci0 + 1)
                return carry
            lax.fori_loop(0, (NCHUNK - 1) // 2, pair, 0)
            gather(0, NCHUNK - 1).wait()
            scatter(0, NCHUNK - 1)
            plsc.subcore_barrier()

            for b in range(RPS // ZR):
                r0 = s * RPS + b * ZR
                pltpu.sync_copy(aggsh.at[pl.ds(r0, ZR)], oh.at[c, pl.ds(r0, ZR)])

            @pl.when(s == NS - 1)
            def _():
                pltpu.sync_copy(aggsh.at[pl.ds(NS * RPS, 16)],
                                oh.at[c, pl.ds(NS * RPS, 16)])
            plsc.subcore_barrier()

    return k(y0, y1, y2, src0, src1, src2, dst0, dst1, dst2)


def _tc_fuse(Wz, Wr, Wh, Lz1, Lr1, Lh1, bz, br, bh, lzb, lrb, lhb):
    def body(wz, wr, wh, l1z, l1r, l1h, bz_, br_, bh_, lz, lr, lh, wf, bf):
        for g, (w, l1, b_, lb) in enumerate(
                ((wz, l1z, bz_, lz), (wr, l1r, br_, lr), (wh, l1h, bh_, lh))):
            wf[g] = jnp.dot(w[...], l1[...], preferred_element_type=jnp.float32)
            bf[g] = jnp.dot(b_[...], l1[...], preferred_element_type=jnp.float32) + lb[...]

    full = lambda shp: pl.BlockSpec(shp, lambda: tuple(0 for _ in shp))
    return pl.pallas_call(
        body,
        grid=(),
        in_specs=[full((D, D))] * 6 + [full((1, D))] * 6,
        out_specs=[full((3, D, D)), full((3, 1, D))],
        out_shape=[jax.ShapeDtypeStruct((3, D, D), jnp.float32),
                   jax.ShapeDtypeStruct((3, 1, D), jnp.float32)],
    )(Wz, Wr, Wh, Lz1, Lr1, Lh1, bz, br, bh, lzb, lrb, lhb)


def _tc_lin1(x0, x1, x2, degp, W1, b1):
    def body(x0b, x1b, x2b, dgb, w1, b1b, y0b, y1b, y2b, disb):
        w = w1[...]
        for t, (xb, yb) in enumerate(((x0b, y0b), (x1b, y1b), (x2b, y2b))):
            deg = dgb[t, 0, :, 0] + dgb[t, 1, :, 0] + 1.0
            dv = lax.rsqrt(deg)
            disb[t] = dv[:, None]
            yb[...] = (jnp.dot(xb[...], w, preferred_element_type=jnp.float32)
                       + b1b[...]) * dv[:, None]

    row = pl.BlockSpec((BN, D), lambda i: (i, 0))
    return pl.pallas_call(
        body,
        grid=(N // BN,),
        in_specs=[row, row, row,
                  pl.BlockSpec((3, NC, BN, DEGW), lambda i: (0, 0, i, 0)),
                  pl.BlockSpec((D, D), lambda i: (0, 0)),
                  pl.BlockSpec((1, D), lambda i: (0, 0))],
        out_specs=[row, row, row, pl.BlockSpec((3, BN, 1), lambda i: (0, i, 0))],
        out_shape=[jax.ShapeDtypeStruct((N, D), jnp.float32)] * 3
                  + [jax.ShapeDtypeStruct((3, N, 1), jnp.float32)],
    )(x0, x1, x2, degp, W1, b1)


def _tc_cell(st, yt, dist, H, Wf, L2, bf):
    def body(sb, yb, db, hb, wf, l2, bfb, ob):
        agg = (sb[0] + sb[1] + yb[...]) * db[...]
        h = hb[...]
        z = jax.nn.sigmoid(jnp.dot(agg, wf[0], preferred_element_type=jnp.float32)
                           + jnp.dot(h, l2[0], preferred_element_type=jnp.float32)
                           + bfb[0])
        r = jax.nn.sigmoid(jnp.dot(agg, wf[1], preferred_element_type=jnp.float32)
                           + jnp.dot(h, l2[1], preferred_element_type=jnp.float32)
                           + bfb[1])
        ht = jnp.tanh(jnp.dot(agg, wf[2], preferred_element_type=jnp.float32)
                      + jnp.dot(h * r, l2[2], preferred_element_type=jnp.float32)
                      + bfb[2])
        ob[...] = z * h + (1.0 - z) * ht

    row = pl.BlockSpec((BN, D), lambda i: (i, 0))
    return pl.pallas_call(
        body,
        grid=(N // BN,),
        in_specs=[pl.BlockSpec((NC, BN, D), lambda i: (0, i, 0)),
                  row,
                  pl.BlockSpec((BN, 1), lambda i: (i, 0)),
                  row,
                  pl.BlockSpec((3, D, D), lambda i: (0, 0, 0)),
                  pl.BlockSpec((3, D, D), lambda i: (0, 0, 0)),
                  pl.BlockSpec((3, 1, D), lambda i: (0, 0, 0))],
        out_specs=row,
        out_shape=jax.ShapeDtypeStruct((N, D), jnp.float32),
    )(st, yt, dist, H, Wf, L2, bf)


def _tc_pool(H, W2, b2):
    def body(hb, w2, b2b, ob):
        p = jnp.max(hb[...], axis=0, keepdims=True)
        ob[...] = jnp.dot(p, w2[...], preferred_element_type=jnp.float32) + b2b[...]

    full = lambda shp: pl.BlockSpec(shp, lambda: tuple(0 for _ in shp))
    return pl.pallas_call(
        body,
        grid=(),
        in_specs=[full((N, D)), full((D, D_OUT)), full((1, D_OUT))],
        out_specs=full((1, D_OUT)),
        out_shape=jax.ShapeDtypeStruct((1, D_OUT), jnp.float32),
    )(H, W2, b2)


def kernel(x0, x1, x2, edge_index0, edge_index1, edge_index2,
           W1, b1, Wz, bz, LzW, Lzb, Wr, br, LrW, Lrb, Wh, bh, LhW, Lhb, W2, b2):
    srcs = [e[0] for e in (edge_index0, edge_index1, edge_index2)]
    dsts = [e[1] for e in (edge_index0, edge_index1, edge_index2)]
    dsts_ch = [x.reshape(NW, NCHUNK, CH) for x in dsts]

    Wf, bf = _tc_fuse(Wz, Wr, Wh, LzW[:D], LrW[:D], LhW[:D],
                      bz.reshape(1, D), br.reshape(1, D), bh.reshape(1, D),
                      Lzb.reshape(1, D), Lrb.reshape(1, D), Lhb.reshape(1, D))
    L2 = jnp.stack([LzW[D:], LrW[D:], LhW[D:]])

    degp = _sc_degree(*dsts).reshape(3, NC, DR * D, 1)[:, :, :N, :]
    y0, y1, y2, dis = _tc_lin1(x0, x1, x2, degp, W1, b1.reshape(1, D))
    s0, s1, s2 = _sc_aggregate(y0, y1, y2, *srcs, *dsts_ch)

    H = jnp.zeros((N, D), jnp.float32)
    for t, (st, yt) in enumerate(((s0, y0), (s1, y1), (s2, y2))):
        H = _tc_cell(st, yt, dis[t], H, Wf, L2, bf)
    return _tc_pool(H, W2, b2.reshape(1, D_OUT))


# speedup vs baseline: 40.4201x; 1.0397x over previous
import functools

import jax
import jax.numpy as jnp
from jax import lax
from jax.experimental import pallas as pl
from jax.experimental.pallas import tpu as pltpu
from jax.experimental.pallas import tpu_sc as plsc

N = 10000
E = 320000
D = 128
D_OUT = 64

NC = 2
NS = 16
NW = NC * NS
EPT = E // NW
CH = 80
NCHUNK = EPT // CH
RPS = 624
ZR = 208
ZB = 16
BN = 2000
DEGW = 1

_mesh = lambda: plsc.VectorSubcoreMesh(core_axis_name="c", subcore_axis_name="s")


DR = 80


def _sc_degree(dst0, dst1, dst2):
    @functools.partial(
        pl.kernel,
        mesh=_mesh(),
        out_type=jax.ShapeDtypeStruct((3, NC, DR, D), jnp.float32),
        scratch_types=[
            pltpu.VMEM((EPT,), jnp.int32),
            pltpu.VMEM((DR, D), jnp.float32),
            pltpu.VMEM((DR,), jnp.int32),
            pltpu.VMEM((16, D), jnp.float32),
            pltpu.VMEM_SHARED((DR, D), jnp.float32),
        ],
        compiler_params=pltpu.CompilerParams(needs_layout_passes=False),
    )
    def k(d0, d1, d2, out, dstv, degv, iotav, zrows, degsh):
        c = lax.axis_index("c")
        s = lax.axis_index("s")
        wid = s * NC + c
        ones = jnp.ones((16,), jnp.float32)
        zero16 = jnp.zeros((16,), jnp.float32)

        for b in range(DR // 16):
            iotav[pl.ds(b * 16, 16)] = lax.iota(jnp.int32, 16) + b * 16

        def zr(i, carry):
            for j in range(D // 16):
                zrows[i, pl.ds(j * 16, 16)] = zero16
            return carry
        lax.fori_loop(0, 16, zr, 0)

        for t, dh in enumerate((d0, d1, d2)):
            @pl.when(s == 0)
            def _():
                for b in range(DR // 16):
                    pltpu.sync_copy(zrows, degsh.at[pl.ds(b * 16, 16)])

            def zb(i, carry):
                for j in range(D // 16):
                    degv[i, pl.ds(j * 16, 16)] = zero16
                return carry
            lax.fori_loop(0, DR, zb, 0)
            pltpu.sync_copy(dh.at[pl.ds(wid * EPT, EPT)], dstv)

            def ab(i, carry):
                idx = dstv[pl.ds(pl.multiple_of(i * 16, 16), 16)]
                plsc.addupdate_scatter(
                    degv,
                    [lax.shift_right_logical(idx, 7), lax.bitwise_and(idx, 127)],
                    ones)
                return carry
            lax.fori_loop(0, EPT // 16, ab, 0)
            plsc.subcore_barrier()
            pltpu.sync_copy(degv, degsh.at[iotav], add=True)
            plsc.subcore_barrier()

            @pl.when(s < 5)
            def _():
                r0 = pl.multiple_of(s * 16, 16)
                pltpu.sync_copy(degsh.at[pl.ds(r0, 16)],
                                out.at[t, c, pl.ds(r0, 16)])
            plsc.subcore_barrier()

    return k(dst0, dst1, dst2)


def _sc_aggregate(y0, y1, y2, src0, src1, src2, dst0, dst1, dst2):
    @functools.partial(
        pl.kernel,
        mesh=_mesh(),
        out_type=[jax.ShapeDtypeStruct((NC, N, D), jnp.float32)] * 3,
        scratch_types=[
            pltpu.VMEM((EPT,), jnp.int32),
            pltpu.VMEM((NCHUNK, CH), jnp.int32),
            pltpu.VMEM((2, CH, D), jnp.float32),
            pltpu.VMEM((ZB, D), jnp.float32),
            pltpu.VMEM_SHARED((N, D), jnp.float32),
            pltpu.SemaphoreType.DMA,
            pltpu.SemaphoreType.DMA,
            pltpu.SemaphoreType.DMA,
            pltpu.SemaphoreType.DMA,
        ],
        compiler_params=pltpu.CompilerParams(needs_layout_passes=False),
    )
    def k(y0h, y1h, y2h, s0h, s1h, s2h, d0h, d1h, d2h, o0, o1, o2,
          srcv, dstv2, rows, zbuf, aggsh, sem0, sem1, ssem0, ssem1):
        c = lax.axis_index("c")
        s = lax.axis_index("s")
        wid = s * NC + c
        z16 = jnp.zeros((16,), jnp.float32)
        sems = (sem0, sem1)
        ssems = (ssem0, ssem1)

        def zb(i, carry):
            for j in range(D // 16):
                zbuf[i, pl.ds(j * 16, 16)] = z16
            return carry
        lax.fori_loop(0, ZB, zb, 0)
        nzb = (RPS // ZB) + jnp.where(s == NS - 1, 1, 0)

        for t in range(3):
            yh = (y0h, y1h, y2h)[t]
            eh = (s0h, s1h, s2h)[t]
            dh = (d0h, d1h, d2h)[t]
            oh = (o0, o1, o2)[t]

            pltpu.sync_copy(eh.at[pl.ds(wid * EPT, EPT)], srcv)
            pltpu.sync_copy(dh.at[wid], dstv2)

            def zrow(i, carry):
                pltpu.sync_copy(zbuf, aggsh.at[pl.ds(s * RPS + i * ZB, ZB)])
                return carry
            lax.fori_loop(0, nzb, zrow, 0)
            plsc.subcore_barrier()

            def gather(buf, ci):
                idx = srcv.at[pl.ds(pl.multiple_of(ci * CH, 8), CH)]
                return pltpu.make_async_copy(yh.at[idx], rows.at[buf], sems[buf])

            def sc_start(buf, ci):
                pltpu.async_copy(rows.at[buf], aggsh.at[dstv2.at[ci]],
                                 ssems[buf], add=True)

            def sc_wait(buf, ci):
                pltpu.make_async_copy(rows.at[buf], aggsh.at[dstv2.at[ci]],
                                      ssems[buf]).wait()

            gather(0, 0).start()
            gather(1, 1).start()

            def quad(kk, carry):
                ci0 = kk * 2
                gather(0, ci0).wait()
                sc_start(0, ci0)
                gather(1, ci0 + 1).wait()
                sc_start(1, ci0 + 1)
                sc_wait(0, ci0)
                gather(0, ci0 + 2).start()
                sc_wait(1, ci0 + 1)
                gather(1, ci0 + 3).start()
                return carry
            lax.fori_loop(0, (NCHUNK - 3) // 2, quad, 0)
            gather(0, NCHUNK - 3).wait()
            sc_start(0, NCHUNK - 3)
            gather(1, NCHUNK - 2).wait()
            sc_start(1, NCHUNK - 2)
            sc_wait(0, NCHUNK - 3)
            gather(0, NCHUNK - 1).start()
            gather(0, NCHUNK - 1).wait()
            sc_start(0, NCHUNK - 1)
            sc_wait(1, NCHUNK - 2)
            sc_wait(0, NCHUNK - 1)
            plsc.subcore_barrier()

            for b in range(RPS // ZR):
                r0 = s * RPS + b * ZR
                pltpu.sync_copy(aggsh.at[pl.ds(r0, ZR)], oh.at[c, pl.ds(r0, ZR)])

            @pl.when(s == NS - 1)
            def _():
                pltpu.sync_copy(aggsh.at[pl.ds(NS * RPS, 16)],
                                oh.at[c, pl.ds(NS * RPS, 16)])
            plsc.subcore_barrier()

    return k(y0, y1, y2, src0, src1, src2, dst0, dst1, dst2)


def _tc_lin1(x0, x1, x2, degp, W1, b1):
    def body(x0b, x1b, x2b, dgb, w1, b1b, y0b, y1b, y2b, disb):
        w = w1[...]
        for t, (xb, yb) in enumerate(((x0b, y0b), (x1b, y1b), (x2b, y2b))):
            deg = dgb[t, 0, :, 0] + dgb[t, 1, :, 0] + 1.0
            dv = lax.rsqrt(deg)
            disb[t] = dv[:, None]
            yb[...] = (jnp.dot(xb[...], w, preferred_element_type=jnp.float32)
                       + b1b[...]) * dv[:, None]

    row = pl.BlockSpec((BN, D), lambda i: (i, 0))
    return pl.pallas_call(
        body,
        grid=(N // BN,),
        in_specs=[row, row, row,
                  pl.BlockSpec((3, NC, BN, DEGW), lambda i: (0, 0, i, 0)),
                  pl.BlockSpec((D, D), lambda i: (0, 0)),
                  pl.BlockSpec((1, D), lambda i: (0, 0))],
        out_specs=[row, row, row, pl.BlockSpec((3, BN, 1), lambda i: (0, i, 0))],
        out_shape=[jax.ShapeDtypeStruct((N, D), jnp.float32)] * 3
                  + [jax.ShapeDtypeStruct((3, N, 1), jnp.float32)],
    )(x0, x1, x2, degp, W1, b1)


def _tc_cell3(s0, s1, s2, y0, y1, y2, dis, Wz, Wr, Wh,
              Lz1, Lr1, Lh1, Lz2, Lr2, Lh2, bz, br, bh, lzb, lrb, lhb):
    def body(s0b, s1b, s2b, y0b, y1b, y2b, db, wz, wr, wh,
             lz1, lr1, lh1, lz2, lr2, lh2, bz_, br_, bh_, lzb_, lrb_, lhb_, ob):
        dot = lambda a, b: jnp.dot(a, b, preferred_element_type=jnp.float32)
        wzf = dot(wz[...], lz1[...])
        wrf = dot(wr[...], lr1[...])
        whf = dot(wh[...], lh1[...])
        bzf = dot(bz_[...], lz1[...]) + lzb_[...]
        brf = dot(br_[...], lr1[...]) + lrb_[...]
        bhf = dot(bh_[...], lh1[...]) + lhb_[...]
        l2z, l2r, l2h = lz2[...], lr2[...], lh2[...]
        h = jnp.zeros((BN, D), jnp.float32)
        for t, (sb, yb) in enumerate(((s0b, y0b), (s1b, y1b), (s2b, y2b))):
            agg = (sb[0] + sb[1] + yb[...]) * db[t]
            z = jax.nn.sigmoid(dot(agg, wzf) + dot(h, l2z) + bzf)
            r = jax.nn.sigmoid(dot(agg, wrf) + dot(h, l2r) + brf)
            ht = jnp.tanh(dot(agg, whf) + dot(h * r, l2h) + bhf)
            h = z * h + (1.0 - z) * ht
        ob[...] = h

    row = pl.BlockSpec((BN, D), lambda i: (i, 0))
    sblk = pl.BlockSpec((NC, BN, D), lambda i: (0, i, 0))
    wblk = pl.BlockSpec((D, D), lambda i: (0, 0))
    bblk = pl.BlockSpec((1, D), lambda i: (0, 0))
    return pl.pallas_call(
        body,
        grid=(N // BN,),
        in_specs=[sblk, sblk, sblk, row, row, row,
                  pl.BlockSpec((3, BN, 1), lambda i: (0, i, 0))]
                 + [wblk] * 9 + [bblk] * 6,
        out_specs=row,
        out_shape=jax.ShapeDtypeStruct((N, D), jnp.float32),
    )(s0, s1, s2, y0, y1, y2, dis, Wz, Wr, Wh,
      Lz1, Lr1, Lh1, Lz2, Lr2, Lh2, bz, br, bh, lzb, lrb, lhb)


def _tc_pool(H, W2, b2):
    def body(hb, w2, b2b, ob):
        p = jnp.max(hb[...], axis=0, keepdims=True)
        ob[...] = jnp.dot(p, w2[...], preferred_element_type=jnp.float32) + b2b[...]

    full = lambda shp: pl.BlockSpec(shp, lambda: tuple(0 for _ in shp))
    return pl.pallas_call(
        body,
        grid=(),
        in_specs=[full((N, D)), full((D, D_OUT)), full((1, D_OUT))],
        out_specs=full((1, D_OUT)),
        out_shape=jax.ShapeDtypeStruct((1, D_OUT), jnp.float32),
    )(H, W2, b2)


def kernel(x0, x1, x2, edge_index0, edge_index1, edge_index2,
           W1, b1, Wz, bz, LzW, Lzb, Wr, br, LrW, Lrb, Wh, bh, LhW, Lhb, W2, b2):
    srcs = [e[0] for e in (edge_index0, edge_index1, edge_index2)]
    dsts = [e[1] for e in (edge_index0, edge_index1, edge_index2)]
    dsts_ch = [x.reshape(NW, NCHUNK, CH) for x in dsts]

    degp = _sc_degree(*dsts).reshape(3, NC, DR * D, 1)[:, :, :N, :]
    y0, y1, y2, dis = _tc_lin1(x0, x1, x2, degp, W1, b1.reshape(1, D))
    s0, s1, s2 = _sc_aggregate(y0, y1, y2, *srcs, *dsts_ch)
    H = _tc_cell3(s0, s1, s2, y0, y1, y2, dis, Wz, Wr, Wh,
                  LzW[:D], LrW[:D], LhW[:D], LzW[D:], LrW[D:], LhW[D:],
                  bz.reshape(1, D), br.reshape(1, D), bh.reshape(1, D),
                  Lzb.reshape(1, D), Lrb.reshape(1, D), Lhb.reshape(1, D))
    return _tc_pool(H, W2, b2.reshape(1, D_OUT))


# speedup vs baseline: 41.2118x; 1.0196x over previous
import functools

import jax
import jax.numpy as jnp
from jax import lax
from jax.experimental import pallas as pl
from jax.experimental.pallas import tpu as pltpu
from jax.experimental.pallas import tpu_sc as plsc

N = 10000
E = 320000
D = 128
D_OUT = 64

NC = 2
NS = 16
NW = NC * NS
EPT = E // NW
CH = 80
NCHUNK = EPT // CH
RPS = 624
ZR = 208
ZB = 16
BN = 2000
DEGW = 1

_mesh = lambda: plsc.VectorSubcoreMesh(core_axis_name="c", subcore_axis_name="s")


DR = 80


def _sc_degree(dst0, dst1, dst2):
    @functools.partial(
        pl.kernel,
        mesh=_mesh(),
        out_type=jax.ShapeDtypeStruct((3, NC, DR, D), jnp.float32),
        scratch_types=[
            pltpu.VMEM((EPT,), jnp.int32),
            pltpu.VMEM((DR, D), jnp.float32),
            pltpu.VMEM((DR,), jnp.int32),
            pltpu.VMEM((16, D), jnp.float32),
            pltpu.VMEM_SHARED((DR, D), jnp.float32),
        ],
        compiler_params=pltpu.CompilerParams(needs_layout_passes=False),
    )
    def k(d0, d1, d2, out, dstv, degv, iotav, zrows, degsh):
        c = lax.axis_index("c")
        s = lax.axis_index("s")
        wid = s * NC + c
        ones = jnp.ones((16,), jnp.float32)
        zero16 = jnp.zeros((16,), jnp.float32)

        for b in range(DR // 16):
            iotav[pl.ds(b * 16, 16)] = lax.iota(jnp.int32, 16) + b * 16

        def zr(i, carry):
            for j in range(D // 16):
                zrows[i, pl.ds(j * 16, 16)] = zero16
            return carry
        lax.fori_loop(0, 16, zr, 0)

        for t, dh in enumerate((d0, d1, d2)):
            @pl.when(s == 0)
            def _():
                for b in range(DR // 16):
                    pltpu.sync_copy(zrows, degsh.at[pl.ds(b * 16, 16)])

            def zb(i, carry):
                for j in range(D // 16):
                    degv[i, pl.ds(j * 16, 16)] = zero16
                return carry
            lax.fori_loop(0, DR, zb, 0)
            pltpu.sync_copy(dh.at[pl.ds(wid * EPT, EPT)], dstv)

            def ab(i, carry):
                idx = dstv[pl.ds(pl.multiple_of(i * 16, 16), 16)]
                plsc.addupdate_scatter(
                    degv,
                    [lax.shift_right_logical(idx, 7), lax.bitwise_and(idx, 127)],
                    ones)
                return carry
            lax.fori_loop(0, EPT // 16, ab, 0)
            plsc.subcore_barrier()
            pltpu.sync_copy(degv, degsh.at[iotav], add=True)
            plsc.subcore_barrier()

            @pl.when(s < 5)
            def _():
                r0 = pl.multiple_of(s * 16, 16)
                pltpu.sync_copy(degsh.at[pl.ds(r0, 16)],
                                out.at[t, c, pl.ds(r0, 16)])
            plsc.subcore_barrier()

    return k(dst0, dst1, dst2)


def _sc_aggregate(y0, y1, y2, src0, src1, src2, dst0, dst1, dst2):
    @functools.partial(
        pl.kernel,
        mesh=_mesh(),
        out_type=[jax.ShapeDtypeStruct((NC, N, D), jnp.float32)] * 3,
        scratch_types=[
            pltpu.VMEM((EPT,), jnp.int32),
            pltpu.VMEM((NCHUNK, CH), jnp.int32),
            pltpu.VMEM((2, CH, D), jnp.float32),
            pltpu.VMEM((ZB, D), jnp.float32),
            pltpu.VMEM_SHARED((N, D), jnp.float32),
            pltpu.SemaphoreType.DMA,
            pltpu.SemaphoreType.DMA,
            pltpu.SemaphoreType.DMA,
            pltpu.SemaphoreType.DMA,
            pltpu.SemaphoreType.DMA,
        ],
        compiler_params=pltpu.CompilerParams(needs_layout_passes=False),
    )
    def k(y0h, y1h, y2h, s0h, s1h, s2h, d0h, d1h, d2h, o0, o1, o2,
          srcv, dstv2, rows, zbuf, aggsh, sem0, sem1, ssem0, ssem1, zsem):
        c = lax.axis_index("c")
        s = lax.axis_index("s")
        wid = s * NC + c
        z16 = jnp.zeros((16,), jnp.float32)
        sems = (sem0, sem1)
        ssems = (ssem0, ssem1)

        def zb(i, carry):
            for j in range(D // 16):
                zbuf[i, pl.ds(j * 16, 16)] = z16
            return carry
        lax.fori_loop(0, ZB, zb, 0)
        nzb = (RPS // ZB) + jnp.where(s == NS - 1, 1, 0)

        for t in range(3):
            yh = (y0h, y1h, y2h)[t]
            eh = (s0h, s1h, s2h)[t]
            dh = (d0h, d1h, d2h)[t]
            oh = (o0, o1, o2)[t]

            stage_s = pltpu.make_async_copy(eh.at[pl.ds(wid * EPT, EPT)], srcv,
                                            sem0)
            stage_d = pltpu.make_async_copy(dh.at[wid], dstv2, sem1)
            stage_s.start()
            stage_d.start()

            def zrow(i, carry):
                pltpu.async_copy(zbuf, aggsh.at[pl.ds(s * RPS + i * ZB, ZB)],
                                 zsem)
                return carry
            lax.fori_loop(0, nzb, zrow, 0)

            def zdrain(i, carry):
                pltpu.make_async_copy(
                    zbuf, aggsh.at[pl.ds(s * RPS + i * ZB, ZB)], zsem).wait()
                return carry
            lax.fori_loop(0, nzb, zdrain, 0)
            stage_s.wait()
            stage_d.wait()
            plsc.subcore_barrier()

            def gather(buf, ci):
                idx = srcv.at[pl.ds(pl.multiple_of(ci * CH, 8), CH)]
                return pltpu.make_async_copy(yh.at[idx], rows.at[buf], sems[buf])

            def sc_start(buf, ci):
                pltpu.async_copy(rows.at[buf], aggsh.at[dstv2.at[ci]],
                                 ssems[buf], add=True)

            def sc_wait(buf, ci):
                pltpu.make_async_copy(rows.at[buf], aggsh.at[dstv2.at[ci]],
                                      ssems[buf]).wait()

            gather(0, 0).start()
            gather(1, 1).start()

            def quad(kk, carry):
                ci0 = kk * 2
                gather(0, ci0).wait()
                sc_start(0, ci0)
                gather(1, ci0 + 1).wait()
                sc_start(1, ci0 + 1)
                sc_wait(0, ci0)
                gather(0, ci0 + 2).start()
                sc_wait(1, ci0 + 1)
                gather(1, ci0 + 3).start()
                return carry
            lax.fori_loop(0, (NCHUNK - 3) // 2, quad, 0)
            gather(0, NCHUNK - 3).wait()
            sc_start(0, NCHUNK - 3)
            gather(1, NCHUNK - 2).wait()
            sc_start(1, NCHUNK - 2)
            sc_wait(0, NCHUNK - 3)
            gather(0, NCHUNK - 1).start()
            gather(0, NCHUNK - 1).wait()
            sc_start(0, NCHUNK - 1)
            sc_wait(1, NCHUNK - 2)
            sc_wait(0, NCHUNK - 1)
            plsc.subcore_barrier()

            for b in range(RPS // ZR):
                r0 = s * RPS + b * ZR
                pltpu.async_copy(aggsh.at[pl.ds(r0, ZR)],
                                 oh.at[c, pl.ds(r0, ZR)], zsem)

            @pl.when(s == NS - 1)
            def _():
                pltpu.async_copy(aggsh.at[pl.ds(NS * RPS, 16)],
                                 oh.at[c, pl.ds(NS * RPS, 16)], zsem)
            for b in range(RPS // ZR):
                r0 = s * RPS + b * ZR
                pltpu.make_async_copy(aggsh.at[pl.ds(r0, ZR)],
                                      oh.at[c, pl.ds(r0, ZR)], zsem).wait()

            @pl.when(s == NS - 1)
            def _():
                pltpu.make_async_copy(aggsh.at[pl.ds(NS * RPS, 16)],
                                      oh.at[c, pl.ds(NS * RPS, 16)], zsem).wait()
            plsc.subcore_barrier()

    return k(y0, y1, y2, src0, src1, src2, dst0, dst1, dst2)


def _tc_lin1(x0, x1, x2, degp, W1, b1):
    def body(x0b, x1b, x2b, dgb, w1, b1b, y0b, y1b, y2b, disb):
        w = w1[...]
        for t, (xb, yb) in enumerate(((x0b, y0b), (x1b, y1b), (x2b, y2b))):
            deg = dgb[t, 0, :, 0] + dgb[t, 1, :, 0] + 1.0
            dv = lax.rsqrt(deg)
            disb[t] = dv[:, None]
            yb[...] = (jnp.dot(xb[...], w, preferred_element_type=jnp.float32)
                       + b1b[...]) * dv[:, None]

    row = pl.BlockSpec((BN, D), lambda i: (i, 0))
    return pl.pallas_call(
        body,
        grid=(N // BN,),
        in_specs=[row, row, row,
                  pl.BlockSpec((3, NC, BN, DEGW), lambda i: (0, 0, i, 0)),
                  pl.BlockSpec((D, D), lambda i: (0, 0)),
                  pl.BlockSpec((1, D), lambda i: (0, 0))],
        out_specs=[row, row, row, pl.BlockSpec((3, BN, 1), lambda i: (0, i, 0))],
        out_shape=[jax.ShapeDtypeStruct((N, D), jnp.float32)] * 3
                  + [jax.ShapeDtypeStruct((3, N, 1), jnp.float32)],
    )(x0, x1, x2, degp, W1, b1)


def _tc_cell3(s0, s1, s2, y0, y1, y2, dis, Wz, Wr, Wh,
              Lz1, Lr1, Lh1, Lz2, Lr2, Lh2, bz, br, bh, lzb, lrb, lhb):
    def body(s0b, s1b, s2b, y0b, y1b, y2b, db, wz, wr, wh,
             lz1, lr1, lh1, lz2, lr2, lh2, bz_, br_, bh_, lzb_, lrb_, lhb_, ob):
        dot = lambda a, b: jnp.dot(a, b, preferred_element_type=jnp.float32)
        wzf = dot(wz[...], lz1[...])
        wrf = dot(wr[...], lr1[...])
        whf = dot(wh[...], lh1[...])
        bzf = dot(bz_[...], lz1[...]) + lzb_[...]
        brf = dot(br_[...], lr1[...]) + lrb_[...]
        bhf = dot(bh_[...], lh1[...]) + lhb_[...]
        l2z, l2r, l2h = lz2[...], lr2[...], lh2[...]
        h = jnp.zeros((BN, D), jnp.float32)
        for t, (sb, yb) in enumerate(((s0b, y0b), (s1b, y1b), (s2b, y2b))):
            agg = (sb[0] + sb[1] + yb[...]) * db[t]
            z = jax.nn.sigmoid(dot(agg, wzf) + dot(h, l2z) + bzf)
            r = jax.nn.sigmoid(dot(agg, wrf) + dot(h, l2r) + brf)
            ht = jnp.tanh(dot(agg, whf) + dot(h * r, l2h) + bhf)
            h = z * h + (1.0 - z) * ht
        ob[...] = h

    row = pl.BlockSpec((BN, D), lambda i: (i, 0))
    sblk = pl.BlockSpec((NC, BN, D), lambda i: (0, i, 0))
    wblk = pl.BlockSpec((D, D), lambda i: (0, 0))
    bblk = pl.BlockSpec((1, D), lambda i: (0, 0))
    return pl.pallas_call(
        body,
        grid=(N // BN,),
        in_specs=[sblk, sblk, sblk, row, row, row,
                  pl.BlockSpec((3, BN, 1), lambda i: (0, i, 0))]
                 + [wblk] * 9 + [bblk] * 6,
        out_specs=row,
        out_shape=jax.ShapeDtypeStruct((N, D), jnp.float32),
    )(s0, s1, s2, y0, y1, y2, dis, Wz, Wr, Wh,
      Lz1, Lr1, Lh1, Lz2, Lr2, Lh2, bz, br, bh, lzb, lrb, lhb)


def _tc_pool(H, W2, b2):
    def body(hb, w2, b2b, ob):
        p = jnp.max(hb[...], axis=0, keepdims=True)
        ob[...] = jnp.dot(p, w2[...], preferred_element_type=jnp.float32) + b2b[...]

    full = lambda shp: pl.BlockSpec(shp, lambda: tuple(0 for _ in shp))
    return pl.pallas_call(
        body,
        grid=(),
        in_specs=[full((N, D)), full((D, D_OUT)), full((1, D_OUT))],
        out_specs=full((1, D_OUT)),
        out_shape=jax.ShapeDtypeStruct((1, D_OUT), jnp.float32),
    )(H, W2, b2)


def kernel(x0, x1, x2, edge_index0, edge_index1, edge_index2,
           W1, b1, Wz, bz, LzW, Lzb, Wr, br, LrW, Lrb, Wh, bh, LhW, Lhb, W2, b2):
    srcs = [e[0] for e in (edge_index0, edge_index1, edge_index2)]
    dsts = [e[1] for e in (edge_index0, edge_index1, edge_index2)]
    dsts_ch = [x.reshape(NW, NCHUNK, CH) for x in dsts]

    degp = _sc_degree(*dsts).reshape(3, NC, DR * D, 1)[:, :, :N, :]
    y0, y1, y2, dis = _tc_lin1(x0, x1, x2, degp, W1, b1.reshape(1, D))
    s0, s1, s2 = _sc_aggregate(y0, y1, y2, *srcs, *dsts_ch)
    H = _tc_cell3(s0, s1, s2, y0, y1, y2, dis, Wz, Wr, Wh,
                  LzW[:D], LrW[:D], LhW[:D], LzW[D:], LrW[D:], LhW[D:],
                  bz.reshape(1, D), br.reshape(1, D), bh.reshape(1, D),
                  Lzb.reshape(1, D), Lrb.reshape(1, D), Lhb.reshape(1, D))
    return _tc_pool(H, W2, b2.reshape(1, D_OUT))
